# baseline (speedup 1.0000x reference)
import functools

import jax
import jax.numpy as jnp
from jax import lax
from jax.experimental import pallas as pl
from jax.experimental.pallas import tpu as pltpu
from jax.experimental.pallas import tpu_sc as plsc

_NUM_CORES = 2
_NUM_SUBCORES = 16
_NUM_WORKERS = _NUM_CORES * _NUM_SUBCORES

_CHUNK = 256

_UNPACK_PERM = tuple(
    32 * (q // 32) + ((q % 32) // 2 if q % 2 == 0 else 16 + (q % 32 - 1) // 2)
    for q in range(128)
)


def _tc_project_table(table_t, w):
    d, v = table_t.shape
    dm = w.shape[1]
    cb = 25600

    def mm(t_ref, w_ref, out_ref):
        acc = jax.lax.dot_general(
            t_ref[...],
            w_ref[...],
            dimension_numbers=(((0,), (0,)), ((), ())),
            preferred_element_type=jnp.float32,
        )
        out_ref[...] = acc.astype(jnp.bfloat16)

    return pl.pallas_call(
        mm,
        grid=((v + cb - 1) // cb,),
        in_specs=[
            pl.BlockSpec((d, cb), lambda i: (0, i)),
            pl.BlockSpec((d, dm), lambda i: (0, 0)),
        ],
        out_specs=pl.BlockSpec((cb, dm), lambda i: (i, 0)),
        out_shape=jax.ShapeDtypeStruct((v, dm), jnp.bfloat16),
    )(table_t, w)


def _sc_gather(flat_idx, p, dm):
    n = flat_idx.shape[0]
    dw = p.shape[1]
    n_per_w = n // _NUM_WORKERS
    n_chunks = n_per_w // _CHUNK
    mesh = plsc.VectorSubcoreMesh(core_axis_name="c", subcore_axis_name="s")

    @functools.partial(
        pl.kernel,
        out_type=jax.ShapeDtypeStruct((n, dm), jnp.float32),
        mesh=mesh,
        scratch_types=[
            pltpu.VMEM((n_per_w,), jnp.int32),
            pltpu.VMEM((_CHUNK, dw), jnp.int32),
            pltpu.VMEM((_CHUNK, dw), jnp.int32),
            pltpu.VMEM((_CHUNK, dm), jnp.float32),
            pltpu.VMEM((_CHUNK, dm), jnp.float32),
            pltpu.SemaphoreType.DMA,
            pltpu.SemaphoreType.DMA,
            pltpu.SemaphoreType.DMA,
            pltpu.SemaphoreType.DMA,
        ],
        compiler_params=pltpu.CompilerParams(
            use_tc_tiling_on_sc=False, needs_layout_passes=False
        ),
    )
    def gather_kernel(
        idx_hbm, p_hbm, out_hbm, idx_all, bf0, bf1, f0, f1, gs0, gs1, ss0, ss1
    ):
        wid = lax.axis_index("s") * _NUM_CORES + lax.axis_index("c")
        base = wid * n_per_w

        pltpu.sync_copy(idx_hbm.at[pl.ds(base, n_per_w)], idx_all)

        def fire_gather(c, bf, gs):
            pltpu.async_copy(p_hbm.at[idx_all.at[pl.ds(c * _CHUNK, _CHUNK)]], bf, gs)

        def drain_gather(bf, gs):
            pltpu.make_async_copy(p_hbm.at[pl.ds(0, _CHUNK)], bf, gs).wait()

        def fire_store(c, f, ss):
            pltpu.async_copy(f, out_hbm.at[pl.ds(base + c * _CHUNK, _CHUNK)], ss)

        def drain_store(f, ss):
            pltpu.make_async_copy(out_hbm.at[pl.ds(0, _CHUNK)], f, ss).wait()

        def convert(bf, f):
            def conv_row(r, _):
                for g in range(4):
                    v = bf[r, pl.ds(16 * g, 16)]
                    vb = plsc.bitcast(v, jnp.bfloat16)
                    a, b = plsc.unpack(vb, format=plsc.PackFormat.INTERLEAVED)
                    f[r, pl.ds(32 * g, 16)] = a
                    f[r, pl.ds(32 * g + 16, 16)] = b
                return 0

            lax.fori_loop(0, _CHUNK, conv_row, 0)

        def step(j, c, bf, f, gs, ss):
            drain_gather(bf, gs)

            @pl.when(j > 0)
            def _():
                drain_store(f, ss)

            convert(bf, f)

            @pl.when(c + 2 < n_chunks)
            def _():
                fire_gather(c + 2, bf, gs)

            fire_store(c, f, ss)

        fire_gather(0, bf0, gs0)
        fire_gather(1, bf1, gs1)

        def body(j, _):
            step(j, 2 * j, bf0, f0, gs0, ss0)
            step(j, 2 * j + 1, bf1, f1, gs1, ss1)
            return 0

        lax.fori_loop(0, n_chunks // 2, body, 0)

        drain_store(f0, ss0)
        drain_store(f1, ss1)

    return gather_kernel(flat_idx, p)


def kernel(x, table, W):
    b, l = x.shape
    dm = W.shape[1]
    flat_idx = x.reshape(b * l).astype(jnp.int32)
    w_perm = W[:, jnp.array(_UNPACK_PERM, dtype=jnp.int32)]
    p = _tc_project_table(table.T, w_perm)
    v = p.shape[0]
    p_words = jax.lax.bitcast_convert_type(
        p.reshape(v, dm // 2, 2), jnp.int32
    )
    out = _sc_gather(flat_idx, p_words, dm)
    return out.reshape(b, l, dm)


# speedup vs baseline: 7.0347x; 7.0347x over previous
import functools

import jax
import jax.numpy as jnp
from jax import lax
from jax.experimental import pallas as pl
from jax.experimental.pallas import tpu as pltpu
from jax.experimental.pallas import tpu_sc as plsc

_NUM_CORES = 2
_NUM_SUBCORES = 16
_NUM_WORKERS = _NUM_CORES * _NUM_SUBCORES

_CHUNK = 200


def _tc_project_table(table_t, w):
    d, v = table_t.shape
    dm = w.shape[1]
    cb = 25600

    def mm(t_ref, w_ref, out_ref):
        out_ref[...] = jax.lax.dot_general(
            t_ref[...],
            w_ref[...],
            dimension_numbers=(((0,), (0,)), ((), ())),
            preferred_element_type=jnp.float32,
        )

    return pl.pallas_call(
        mm,
        grid=((v + cb - 1) // cb,),
        in_specs=[
            pl.BlockSpec((d, cb), lambda i: (0, i)),
            pl.BlockSpec((d, dm), lambda i: (0, 0)),
        ],
        out_specs=pl.BlockSpec((cb, dm), lambda i: (i, 0)),
        out_shape=jax.ShapeDtypeStruct((v, dm), jnp.float32),
    )(table_t, w)


def _sc_gather(flat_idx, p):
    n = flat_idx.shape[0]
    dm = p.shape[1]
    n_per_w = n // _NUM_WORKERS
    n_chunks = n_per_w // _CHUNK
    mesh = plsc.VectorSubcoreMesh(core_axis_name="c", subcore_axis_name="s")

    @functools.partial(
        pl.kernel,
        out_type=jax.ShapeDtypeStruct((n, dm), jnp.float32),
        mesh=mesh,
        scratch_types=[
            pltpu.VMEM((n_per_w,), jnp.int32),
            pltpu.VMEM((_CHUNK, dm), jnp.float32),
            pltpu.VMEM((_CHUNK, dm), jnp.float32),
            pltpu.VMEM((_CHUNK, dm), jnp.float32),
            pltpu.VMEM((_CHUNK, dm), jnp.float32),
            pltpu.SemaphoreType.DMA,
            pltpu.SemaphoreType.DMA,
            pltpu.SemaphoreType.DMA,
            pltpu.SemaphoreType.DMA,
            pltpu.SemaphoreType.DMA,
            pltpu.SemaphoreType.DMA,
            pltpu.SemaphoreType.DMA,
            pltpu.SemaphoreType.DMA,
        ],
        compiler_params=pltpu.CompilerParams(use_tc_tiling_on_sc=True),
    )
    def gather_kernel(
        idx_hbm, p_hbm, out_hbm, idx_all,
        rows0, rows1, rows2, rows3,
        gs0, gs1, gs2, gs3, ss0, ss1, ss2, ss3,
    ):
        wid = lax.axis_index("s") * _NUM_CORES + lax.axis_index("c")
        base = wid * n_per_w
        bufs = (rows0, rows1, rows2, rows3)
        gsems = (gs0, gs1, gs2, gs3)
        ssems = (ss0, ss1, ss2, ss3)

        pltpu.sync_copy(idx_hbm.at[pl.ds(base, n_per_w)], idx_all)

        def fire_gather(c, k):
            pltpu.async_copy(
                p_hbm.at[idx_all.at[pl.ds(c * _CHUNK, _CHUNK)]], bufs[k], gsems[k]
            )

        def drain_gather(k):
            pltpu.make_async_copy(p_hbm.at[pl.ds(0, _CHUNK)], bufs[k], gsems[k]).wait()

        def fire_store(c, k):
            pltpu.async_copy(bufs[k], out_hbm.at[pl.ds(base + c * _CHUNK, _CHUNK)], ssems[k])

        def drain_store(k):
            pltpu.make_async_copy(out_hbm.at[pl.ds(0, _CHUNK)], bufs[k], ssems[k]).wait()

        fire_gather(0, 0)
        fire_gather(1, 1)

        def step(c, k, kn):
            drain_gather(k)
            fire_store(c, k)

            @pl.when(c >= 2)
            def _():
                drain_store(kn)

            @pl.when(c + 2 < n_chunks)
            def _():
                fire_gather(c + 2, kn)

        def body(j, _):
            c0 = 4 * j
            for k in range(4):
                step(c0 + k, k, (k + 2) % 4)
            return 0

        lax.fori_loop(0, n_chunks // 4, body, 0)

        drain_store((n_chunks - 2) % 4)
        drain_store((n_chunks - 1) % 4)

    return gather_kernel(flat_idx, p)


def kernel(x, table, W):
    b, l = x.shape
    dm = W.shape[1]
    flat_idx = x.reshape(b * l).astype(jnp.int32)
    p = _tc_project_table(table.T, W)
    out = _sc_gather(flat_idx, p)
    return out.reshape(b, l, dm)


# speedup vs baseline: 7.0586x; 1.0034x over previous
import functools

import jax
import jax.numpy as jnp
from jax import lax
from jax.experimental import pallas as pl
from jax.experimental.pallas import tpu as pltpu
from jax.experimental.pallas import tpu_sc as plsc

_NUM_CORES = 2
_NUM_SUBCORES = 16
_NUM_WORKERS = _NUM_CORES * _NUM_SUBCORES

_CHUNK = 400


def _tc_project_table(table_t, w):
    d, v = table_t.shape
    dm = w.shape[1]
    cb = 25600

    def mm(t_ref, w_ref, out_ref):
        out_ref[...] = jax.lax.dot_general(
            t_ref[...],
            w_ref[...],
            dimension_numbers=(((0,), (0,)), ((), ())),
            preferred_element_type=jnp.float32,
        )

    return pl.pallas_call(
        mm,
        grid=((v + cb - 1) // cb,),
        in_specs=[
            pl.BlockSpec((d, cb), lambda i: (0, i)),
            pl.BlockSpec((d, dm), lambda i: (0, 0)),
        ],
        out_specs=pl.BlockSpec((cb, dm), lambda i: (i, 0)),
        out_shape=jax.ShapeDtypeStruct((v, dm), jnp.float32),
    )(table_t, w)


def _sc_gather(flat_idx, p):
    n = flat_idx.shape[0]
    dm = p.shape[1]
    n_per_w = n // _NUM_WORKERS
    n_chunks = n_per_w // _CHUNK
    mesh = plsc.VectorSubcoreMesh(core_axis_name="c", subcore_axis_name="s")

    @functools.partial(
        pl.kernel,
        out_type=jax.ShapeDtypeStruct((n, dm), jnp.float32),
        mesh=mesh,
        scratch_types=[
            pltpu.VMEM((n_per_w,), jnp.int32),
            pltpu.VMEM((_CHUNK, dm), jnp.float32),
            pltpu.VMEM((_CHUNK, dm), jnp.float32),
            pltpu.SemaphoreType.DMA,
            pltpu.SemaphoreType.DMA,
        ],
        compiler_params=pltpu.CompilerParams(use_tc_tiling_on_sc=True),
    )
    def gather_kernel(idx_hbm, p_hbm, out_hbm, idx_all, rows0, rows1, sem0, sem1):
        wid = lax.axis_index("s") * _NUM_CORES + lax.axis_index("c")
        base = wid * n_per_w

        pltpu.sync_copy(idx_hbm.at[pl.ds(base, n_per_w)], idx_all)

        def fire(c, rows, sem):
            pltpu.async_copy(p_hbm.at[idx_all.at[pl.ds(c * _CHUNK, _CHUNK)]], rows, sem)

        def drain(rows, sem):
            pltpu.make_async_copy(p_hbm.at[pl.ds(0, _CHUNK)], rows, sem).wait()

        def store(c, rows):
            pltpu.sync_copy(rows, out_hbm.at[pl.ds(base + c * _CHUNK, _CHUNK)])

        fire(0, rows0, sem0)

        def body(j, _):
            c0 = 2 * j
            fire(c0 + 1, rows1, sem1)
            drain(rows0, sem0)
            store(c0, rows0)
            fire(c0 + 2, rows0, sem0)
            drain(rows1, sem1)
            store(c0 + 1, rows1)
            return 0

        lax.fori_loop(0, n_chunks // 2 - 1, body, 0)

        c_last = n_chunks - 2
        fire(c_last + 1, rows1, sem1)
        drain(rows0, sem0)
        store(c_last, rows0)
        drain(rows1, sem1)
        store(c_last + 1, rows1)

    return gather_kernel(flat_idx, p)


def kernel(x, table, W):
    b, l = x.shape
    dm = W.shape[1]
    flat_idx = x.reshape(b * l).astype(jnp.int32)
    p = _tc_project_table(table.T, W)
    out = _sc_gather(flat_idx, p)
    return out.reshape(b, l, dm)
